# Initial kernel scaffold; baseline (speedup 1.0000x reference)
import jax
import jax.numpy as jnp
from jax.experimental import pallas as pl


def kernel(x, edge_index, W_l, b_l, W_r):
    raise NotImplementedError("write your pallas kernel here")


# speedup vs baseline: 7.5425x; 7.5425x over previous
import functools

import jax
import jax.numpy as jnp
from jax import lax
from jax.experimental import pallas as pl
from jax.experimental.pallas import tpu as pltpu
from jax.experimental.pallas import tpu_sc as plsc

N_NODES = 10000
N_EDGES = 320000
D = 128

NC = 2
NS = 16
NW = NC * NS
EDGES_PER_WORKER = N_EDGES // NW
CHUNK = 80
NCHUNK = EDGES_PER_WORKER // CHUNK
N_PAD = 10240
ROWS_PER_TILE = N_PAD // NS

_sc_mesh = plsc.VectorSubcoreMesh(core_axis_name="c", subcore_axis_name="s")


@functools.partial(
    pl.kernel,
    out_type=jax.ShapeDtypeStruct((NC, N_PAD, D), jnp.float32),
    mesh=_sc_mesh,
    scratch_types=[
        pltpu.VMEM((NCHUNK, CHUNK), jnp.int32),
        pltpu.VMEM((NCHUNK, CHUNK), jnp.int32),
        pltpu.VMEM((CHUNK, D), jnp.float32),
        pltpu.VMEM_SHARED((N_PAD, D), jnp.float32),
        pltpu.SemaphoreType.DMA,
    ],
)
def _sc_aggregate(x_hbm, src_hbm, dst_hbm, zeros_hbm, out_hbm,
                  src_v, dst_v, rows_v, acc, sem):
    c = lax.axis_index("c")
    s = lax.axis_index("s")
    w = c * NS + s

    r0 = pl.multiple_of(s * ROWS_PER_TILE, 8)
    pltpu.sync_copy(zeros_hbm.at[pl.ds(r0, ROWS_PER_TILE)],
                    acc.at[pl.ds(r0, ROWS_PER_TILE)])
    pltpu.sync_copy(src_hbm.at[w], src_v)
    pltpu.sync_copy(dst_hbm.at[w], dst_v)
    plsc.subcore_barrier()

    def body(j, carry):
        pltpu.async_copy(x_hbm.at[src_v.at[j]], rows_v, sem).wait()
        pltpu.sync_copy(rows_v, acc.at[dst_v.at[j]], add=True)
        return carry

    lax.fori_loop(0, NCHUNK, body, 0)
    plsc.subcore_barrier()

    pltpu.sync_copy(acc.at[pl.ds(r0, ROWS_PER_TILE)],
                    out_hbm.at[c].at[pl.ds(r0, ROWS_PER_TILE)])


_ROW_BLK = 1000


def _tc_combine(p_ref, x_ref, wl_ref, wr_ref, b_ref, o_ref):
    agg = p_ref[0] + p_ref[1]
    y = jnp.dot(agg, wl_ref[...], preferred_element_type=jnp.float32)
    y = y + jnp.dot(x_ref[...], wr_ref[...], preferred_element_type=jnp.float32)
    o_ref[...] = jnp.tanh(y + b_ref[...])


_tc_call = pl.pallas_call(
    _tc_combine,
    grid=(N_NODES // _ROW_BLK,),
    in_specs=[
        pl.BlockSpec((NC, _ROW_BLK, D), lambda i: (0, i, 0)),
        pl.BlockSpec((_ROW_BLK, D), lambda i: (i, 0)),
        pl.BlockSpec((D, D), lambda i: (0, 0)),
        pl.BlockSpec((D, D), lambda i: (0, 0)),
        pl.BlockSpec((1, D), lambda i: (0, 0)),
    ],
    out_specs=pl.BlockSpec((_ROW_BLK, D), lambda i: (i, 0)),
    out_shape=jax.ShapeDtypeStruct((N_NODES, D), jnp.float32),
)


def kernel(x, edge_index, W_l, b_l, W_r):
    src = edge_index[0].astype(jnp.int32).reshape(NW, NCHUNK, CHUNK)
    dst = edge_index[1].astype(jnp.int32).reshape(NW, NCHUNK, CHUNK)
    zeros = jnp.zeros((N_PAD, D), jnp.float32)
    partials = _sc_aggregate(x, src, dst, zeros)
    return _tc_call(partials, x, W_l.T, W_r.T, b_l.reshape(1, D))


# speedup vs baseline: 11.4817x; 1.5223x over previous
import functools

import jax
import jax.numpy as jnp
from jax import lax
from jax.experimental import pallas as pl
from jax.experimental.pallas import tpu as pltpu
from jax.experimental.pallas import tpu_sc as plsc

N_NODES = 10000
N_EDGES = 320000
D = 128

NC = 2
NS = 16
NW = NC * NS
EDGES_PER_WORKER = N_EDGES // NW
CHUNK = 80
EPW_PAD = 10080
NCHUNK = EPW_PAD // CHUNK
PAD_PER_W = EPW_PAD - EDGES_PER_WORKER
N_PAD = 10240
ROWS_PER_TILE = N_PAD // NS

_sc_mesh = plsc.VectorSubcoreMesh(core_axis_name="c", subcore_axis_name="s")


@functools.partial(
    pl.kernel,
    out_type=jax.ShapeDtypeStruct((NC, N_PAD, D), jnp.float32),
    mesh=_sc_mesh,
    scratch_types=[
        pltpu.VMEM((EPW_PAD,), jnp.int32),
        pltpu.VMEM((NCHUNK, CHUNK), jnp.int32),
        pltpu.VMEM((CHUNK, D), jnp.float32),
        pltpu.VMEM((CHUNK, D), jnp.float32),
        pltpu.VMEM_SHARED((N_PAD, D), jnp.float32),
        pltpu.SemaphoreType.DMA,
    ],
)
def _sc_aggregate(x_hbm, src_hbm, dst_hbm, zeros_hbm, out_hbm,
                  src_v, dst_v, rows0_v, rows1_v, acc, sem):
    c = lax.axis_index("c")
    s = lax.axis_index("s")
    w = c * NS + s

    r0 = pl.multiple_of(s * ROWS_PER_TILE, 8)
    pltpu.sync_copy(zeros_hbm.at[pl.ds(r0, ROWS_PER_TILE)],
                    acc.at[pl.ds(r0, ROWS_PER_TILE)])
    e0 = pl.multiple_of(w * EPW_PAD, 8)
    pltpu.sync_copy(src_hbm.at[pl.ds(e0, EPW_PAD)], src_v)
    pltpu.sync_copy(dst_hbm.at[w], dst_v)
    plsc.subcore_barrier()

    bufs = (rows0_v, rows1_v)

    def start_gather(j, b):
        pltpu.async_copy(
            x_hbm.at[src_v.at[pl.ds(pl.multiple_of(j * CHUNK, 8), CHUNK)]],
            bufs[b], sem)

    def wait_gather(b):
        pltpu.make_async_copy(x_hbm.at[src_v.at[pl.ds(0, CHUNK)]],
                              bufs[b], sem).wait()

    def scatter(j, b):
        pltpu.sync_copy(bufs[b], acc.at[dst_v.at[j]], add=True)

    start_gather(0, 0)
    start_gather(1, 1)

    def body(i, carry):
        j = 2 * i
        wait_gather(0)
        scatter(j, 0)
        start_gather(j + 2, 0)
        wait_gather(1)
        scatter(j + 1, 1)
        start_gather(j + 3, 1)
        return carry

    lax.fori_loop(0, NCHUNK // 2 - 1, body, 0)
    wait_gather(0)
    scatter(NCHUNK - 2, 0)
    wait_gather(1)
    scatter(NCHUNK - 1, 1)
    plsc.subcore_barrier()

    pltpu.sync_copy(acc.at[pl.ds(r0, ROWS_PER_TILE)],
                    out_hbm.at[c].at[pl.ds(r0, ROWS_PER_TILE)])


_ROW_BLK = 1000


def _tc_combine(p_ref, x_ref, wl_ref, wr_ref, b_ref, o_ref):
    agg = p_ref[0] + p_ref[1]
    y = jnp.dot(agg, wl_ref[...], preferred_element_type=jnp.float32)
    y = y + jnp.dot(x_ref[...], wr_ref[...], preferred_element_type=jnp.float32)
    o_ref[...] = jnp.tanh(y + b_ref[...])


_tc_call = pl.pallas_call(
    _tc_combine,
    grid=(N_NODES // _ROW_BLK,),
    in_specs=[
        pl.BlockSpec((NC, _ROW_BLK, D), lambda i: (0, i, 0)),
        pl.BlockSpec((_ROW_BLK, D), lambda i: (i, 0)),
        pl.BlockSpec((D, D), lambda i: (0, 0)),
        pl.BlockSpec((D, D), lambda i: (0, 0)),
        pl.BlockSpec((1, D), lambda i: (0, 0)),
    ],
    out_specs=pl.BlockSpec((_ROW_BLK, D), lambda i: (i, 0)),
    out_shape=jax.ShapeDtypeStruct((N_NODES, D), jnp.float32),
)


def kernel(x, edge_index, W_l, b_l, W_r):
    pad_ids = jnp.arange(NW * PAD_PER_W, dtype=jnp.int32).reshape(NW, PAD_PER_W)
    src = jnp.concatenate(
        [edge_index[0].astype(jnp.int32).reshape(NW, -1), pad_ids % N_NODES],
        axis=1).reshape(-1)
    dst = jnp.concatenate(
        [edge_index[1].astype(jnp.int32).reshape(NW, -1),
         N_NODES + pad_ids % (N_PAD - N_NODES)],
        axis=1).reshape(NW, NCHUNK, CHUNK)
    zeros = jnp.zeros((N_PAD, D), jnp.float32)
    partials = _sc_aggregate(x, src, dst, zeros)
    return _tc_call(partials, x, W_l.T, W_r.T, b_l.reshape(1, D))


# speedup vs baseline: 11.8638x; 1.0333x over previous
import functools

import jax
import jax.numpy as jnp
from jax import lax
from jax.experimental import pallas as pl
from jax.experimental.pallas import tpu as pltpu
from jax.experimental.pallas import tpu_sc as plsc

N_NODES = 10000
N_EDGES = 320000
D = 128

NC = 2
NS = 16
NW = NC * NS
EDGES_PER_WORKER = N_EDGES // NW
CHUNK = 80
EPW_PAD = 10080
NCHUNK = EPW_PAD // CHUNK
PAD_PER_W = EPW_PAD - EDGES_PER_WORKER
N_PAD = 10240
ROWS_PER_TILE = N_PAD // NS

_sc_mesh = plsc.VectorSubcoreMesh(core_axis_name="c", subcore_axis_name="s")


@functools.partial(
    pl.kernel,
    out_type=jax.ShapeDtypeStruct((NC, N_PAD, D), jnp.float32),
    mesh=_sc_mesh,
    scratch_types=[
        pltpu.VMEM((EPW_PAD,), jnp.int32),
        pltpu.VMEM((NCHUNK, CHUNK), jnp.int32),
        pltpu.VMEM((CHUNK, D), jnp.float32),
        pltpu.VMEM((CHUNK, D), jnp.float32),
        pltpu.VMEM_SHARED((N_PAD, D), jnp.float32),
        pltpu.SemaphoreType.DMA,
    ],
)
def _sc_aggregate(x_hbm, src_hbm, dst_hbm, out_hbm,
                  src_v, dst_v, rows0_v, rows1_v, acc, sem):
    c = lax.axis_index("c")
    s = lax.axis_index("s")
    w = c * NS + s

    def zrow(i, carry):
        for k in range(D // 16):
            rows0_v[i, pl.ds(k * 16, 16)] = jnp.zeros((16,), jnp.float32)
        return carry

    lax.fori_loop(0, CHUNK, zrow, 0)
    r0 = pl.multiple_of(s * ROWS_PER_TILE, 8)
    for t in range(ROWS_PER_TILE // CHUNK):
        pltpu.sync_copy(rows0_v,
                        acc.at[pl.ds(r0 + t * CHUNK, CHUNK)])
    e0 = pl.multiple_of(w * EPW_PAD, 8)
    pltpu.sync_copy(src_hbm.at[pl.ds(e0, EPW_PAD)], src_v)
    pltpu.sync_copy(dst_hbm.at[w], dst_v)
    plsc.subcore_barrier()

    bufs = (rows0_v, rows1_v)

    def start_gather(j, b):
        pltpu.async_copy(
            x_hbm.at[src_v.at[pl.ds(pl.multiple_of(j * CHUNK, 8), CHUNK)]],
            bufs[b], sem)

    def wait_gather(b):
        pltpu.make_async_copy(x_hbm.at[src_v.at[pl.ds(0, CHUNK)]],
                              bufs[b], sem).wait()

    def scatter(j, b):
        pltpu.sync_copy(bufs[b], acc.at[dst_v.at[j]], add=True)

    start_gather(0, 0)
    start_gather(1, 1)

    def body(i, carry):
        j = 2 * i
        for b in range(2):
            wait_gather(b)
            scatter(j + b, b)
            start_gather(j + b + 2, b)
        return carry

    lax.fori_loop(0, NCHUNK // 2 - 1, body, 0)
    for b in range(2):
        wait_gather(b)
        scatter(NCHUNK - 2 + b, b)
    plsc.subcore_barrier()

    pltpu.sync_copy(acc.at[pl.ds(r0, ROWS_PER_TILE)],
                    out_hbm.at[c].at[pl.ds(r0, ROWS_PER_TILE)])


_ROW_BLK = 1000


_DN_T = (((1,), (1,)), ((), ()))


def _tc_combine(p_ref, x_ref, wl_ref, wr_ref, b_ref, o_ref):
    agg = p_ref[0] + p_ref[1]
    y = lax.dot_general(agg, wl_ref[...], _DN_T,
                        preferred_element_type=jnp.float32)
    y = y + lax.dot_general(x_ref[...], wr_ref[...], _DN_T,
                            preferred_element_type=jnp.float32)
    o_ref[...] = jnp.tanh(y + b_ref[...])


_tc_call = pl.pallas_call(
    _tc_combine,
    grid=(N_NODES // _ROW_BLK,),
    in_specs=[
        pl.BlockSpec((NC, _ROW_BLK, D), lambda i: (0, i, 0)),
        pl.BlockSpec((_ROW_BLK, D), lambda i: (i, 0)),
        pl.BlockSpec((D, D), lambda i: (0, 0)),
        pl.BlockSpec((D, D), lambda i: (0, 0)),
        pl.BlockSpec((1, D), lambda i: (0, 0)),
    ],
    out_specs=pl.BlockSpec((_ROW_BLK, D), lambda i: (i, 0)),
    out_shape=jax.ShapeDtypeStruct((N_NODES, D), jnp.float32),
)


def kernel(x, edge_index, W_l, b_l, W_r):
    pad_ids = jnp.arange(NW * PAD_PER_W, dtype=jnp.int32).reshape(NW, PAD_PER_W)
    src = jnp.concatenate(
        [edge_index[0].astype(jnp.int32).reshape(NW, -1), pad_ids % N_NODES],
        axis=1).reshape(-1)
    dst = jnp.concatenate(
        [edge_index[1].astype(jnp.int32).reshape(NW, -1),
         N_NODES + pad_ids % (N_PAD - N_NODES)],
        axis=1).reshape(NW, NCHUNK, CHUNK)
    partials = _sc_aggregate(x, src, dst)
    return _tc_call(partials, x, W_l, W_r, b_l.reshape(1, D))


# speedup vs baseline: 11.9271x; 1.0053x over previous
import functools

import jax
import jax.numpy as jnp
from jax import lax
from jax.experimental import pallas as pl
from jax.experimental.pallas import tpu as pltpu
from jax.experimental.pallas import tpu_sc as plsc

N_NODES = 10000
N_EDGES = 320000
D = 128

NC = 2
NS = 16
NW = NC * NS
EPW = N_EDGES // NW
CHUNK = 80
NCHUNK = EPW // CHUNK
N_PAD = 10240
ROWS_PER_TILE = N_PAD // NS

_sc_mesh = plsc.VectorSubcoreMesh(core_axis_name="c", subcore_axis_name="s")


@functools.partial(
    pl.kernel,
    out_type=jax.ShapeDtypeStruct((NC, N_PAD, D), jnp.float32),
    mesh=_sc_mesh,
    scratch_types=[
        pltpu.VMEM((EPW,), jnp.int32),
        pltpu.VMEM((NCHUNK, CHUNK), jnp.int32),
        pltpu.VMEM((CHUNK, D), jnp.float32),
        pltpu.VMEM((CHUNK, D), jnp.float32),
        pltpu.VMEM_SHARED((N_PAD, D), jnp.float32),
        pltpu.SemaphoreType.DMA,
    ],
)
def _sc_aggregate(x_hbm, src_hbm, dst_hbm, out_hbm,
                  src_v, dst_v, rows0_v, rows1_v, acc, sem):
    c = lax.axis_index("c")
    s = lax.axis_index("s")
    w = c * NS + s

    def zrow(i, carry):
        for k in range(D // 16):
            rows0_v[i, pl.ds(k * 16, 16)] = jnp.zeros((16,), jnp.float32)
        return carry

    lax.fori_loop(0, CHUNK, zrow, 0)
    r0 = pl.multiple_of(s * ROWS_PER_TILE, 8)
    for t in range(ROWS_PER_TILE // CHUNK):
        pltpu.sync_copy(rows0_v,
                        acc.at[pl.ds(r0 + t * CHUNK, CHUNK)])
    e0 = pl.multiple_of(w * EPW, 8)
    pltpu.sync_copy(src_hbm.at[pl.ds(e0, EPW)], src_v)
    pltpu.sync_copy(dst_hbm.at[w], dst_v)
    plsc.subcore_barrier()

    bufs = (rows0_v, rows1_v)

    def start_gather(j, b):
        pltpu.async_copy(
            x_hbm.at[src_v.at[pl.ds(pl.multiple_of(j * CHUNK, 8), CHUNK)]],
            bufs[b], sem)

    def wait_gather(b):
        pltpu.make_async_copy(x_hbm.at[src_v.at[pl.ds(0, CHUNK)]],
                              bufs[b], sem).wait()

    def scatter(j, b):
        pltpu.sync_copy(bufs[b], acc.at[dst_v.at[j]], add=True)

    start_gather(0, 0)
    start_gather(1, 1)

    def body(i, carry):
        j = 2 * i
        for b in range(2):
            wait_gather(b)
            scatter(j + b, b)
            start_gather(j + b + 2, b)
        return carry

    lax.fori_loop(0, NCHUNK // 2 - 1, body, 0)
    wait_gather(0)
    scatter(NCHUNK - 3, 0)
    start_gather(NCHUNK - 1, 0)
    wait_gather(1)
    scatter(NCHUNK - 2, 1)
    wait_gather(0)
    scatter(NCHUNK - 1, 0)
    plsc.subcore_barrier()

    pltpu.sync_copy(acc.at[pl.ds(r0, ROWS_PER_TILE)],
                    out_hbm.at[c].at[pl.ds(r0, ROWS_PER_TILE)])


_ROW_BLK = 1000
_DN_T = (((1,), (1,)), ((), ()))


def _tc_root(x_ref, wr_ref, b_ref, o_ref):
    o_ref[...] = lax.dot_general(
        x_ref[...], wr_ref[...], _DN_T,
        preferred_element_type=jnp.float32) + b_ref[...]


_tc_root_call = pl.pallas_call(
    _tc_root,
    grid=(N_NODES // _ROW_BLK,),
    in_specs=[
        pl.BlockSpec((_ROW_BLK, D), lambda i: (i, 0)),
        pl.BlockSpec((D, D), lambda i: (0, 0)),
        pl.BlockSpec((1, D), lambda i: (0, 0)),
    ],
    out_specs=pl.BlockSpec((_ROW_BLK, D), lambda i: (i, 0)),
    out_shape=jax.ShapeDtypeStruct((N_NODES, D), jnp.float32),
)


def _tc_fin(p_ref, r_ref, wl_ref, o_ref):
    agg = p_ref[0] + p_ref[1]
    y = lax.dot_general(agg, wl_ref[...], _DN_T,
                        preferred_element_type=jnp.float32)
    o_ref[...] = jnp.tanh(y + r_ref[...])


_tc_fin_call = pl.pallas_call(
    _tc_fin,
    grid=(N_NODES // _ROW_BLK,),
    in_specs=[
        pl.BlockSpec((NC, _ROW_BLK, D), lambda i: (0, i, 0)),
        pl.BlockSpec((_ROW_BLK, D), lambda i: (i, 0)),
        pl.BlockSpec((D, D), lambda i: (0, 0)),
    ],
    out_specs=pl.BlockSpec((_ROW_BLK, D), lambda i: (i, 0)),
    out_shape=jax.ShapeDtypeStruct((N_NODES, D), jnp.float32),
)


def kernel(x, edge_index, W_l, b_l, W_r):
    src = edge_index[0].astype(jnp.int32)
    dst = edge_index[1].astype(jnp.int32).reshape(NW, NCHUNK, CHUNK)
    root = _tc_root_call(x, W_r, b_l.reshape(1, D))
    partials = _sc_aggregate(x, src, dst)
    return _tc_fin_call(partials, root, W_l)


# speedup vs baseline: 13.0822x; 1.0968x over previous
import functools

import jax
import jax.numpy as jnp
from jax import lax
from jax.experimental import pallas as pl
from jax.experimental.pallas import tpu as pltpu
from jax.experimental.pallas import tpu_sc as plsc

N_NODES = 10000
N_EDGES = 320000
D = 128

NC = 2
NS = 16
NW = NC * NS
EPW = N_EDGES // NW
CHUNK = 80
NCHUNK = EPW // CHUNK
N_PAD = 10240
ROWS_PER_TILE = N_PAD // NS

_sc_mesh = plsc.VectorSubcoreMesh(core_axis_name="c", subcore_axis_name="s")


@functools.partial(
    pl.kernel,
    out_type=jax.ShapeDtypeStruct((NC, N_PAD, D), jnp.float32),
    mesh=_sc_mesh,
    scratch_types=[
        pltpu.VMEM((EPW,), jnp.int32),
        pltpu.VMEM((EPW,), jnp.int32),
        pltpu.VMEM((CHUNK, D), jnp.float32),
        pltpu.VMEM((CHUNK, D), jnp.float32),
        pltpu.VMEM_SHARED((N_PAD, D), jnp.float32),
        pltpu.SemaphoreType.DMA,
    ],
)
def _sc_aggregate(x_hbm, ei_hbm, out_hbm,
                  src_v, dst_v, rows0_v, rows1_v, acc, sem):
    c = lax.axis_index("c")
    s = lax.axis_index("s")
    w = c * NS + s

    def zrow(i, carry):
        for k in range(D // 16):
            rows0_v[i, pl.ds(k * 16, 16)] = jnp.zeros((16,), jnp.float32)
        return carry

    lax.fori_loop(0, CHUNK, zrow, 0)
    r0 = pl.multiple_of(s * ROWS_PER_TILE, 8)
    for t in range(ROWS_PER_TILE // CHUNK):
        pltpu.sync_copy(rows0_v,
                        acc.at[pl.ds(r0 + t * CHUNK, CHUNK)])
    e0 = pl.multiple_of(w * EPW, 8)
    pltpu.sync_copy(ei_hbm.at[pl.ds(e0, EPW)], src_v)
    pltpu.sync_copy(ei_hbm.at[pl.ds(N_EDGES + e0, EPW)], dst_v)
    plsc.subcore_barrier()

    bufs = (rows0_v, rows1_v)

    def start_gather(j, b):
        pltpu.async_copy(
            x_hbm.at[src_v.at[pl.ds(pl.multiple_of(j * CHUNK, 8), CHUNK)]],
            bufs[b], sem)

    def wait_gather(b):
        pltpu.make_async_copy(x_hbm.at[src_v.at[pl.ds(0, CHUNK)]],
                              bufs[b], sem).wait()

    def scatter(j, b):
        pltpu.sync_copy(
            bufs[b],
            acc.at[dst_v.at[pl.ds(pl.multiple_of(j * CHUNK, 8), CHUNK)]],
            add=True)

    start_gather(0, 0)
    start_gather(1, 1)

    def body(i, carry):
        j = 2 * i
        for b in range(2):
            wait_gather(b)
            scatter(j + b, b)
            start_gather(j + b + 2, b)
        return carry

    lax.fori_loop(0, NCHUNK // 2 - 1, body, 0)
    wait_gather(0)
    scatter(NCHUNK - 3, 0)
    start_gather(NCHUNK - 1, 0)
    wait_gather(1)
    scatter(NCHUNK - 2, 1)
    wait_gather(0)
    scatter(NCHUNK - 1, 0)
    plsc.subcore_barrier()

    pltpu.sync_copy(acc.at[pl.ds(r0, ROWS_PER_TILE)],
                    out_hbm.at[c].at[pl.ds(r0, ROWS_PER_TILE)])


_ROW_BLK = 1000
_DN_T = (((1,), (1,)), ((), ()))


def _tc_root(x_ref, wr_ref, b_ref, o_ref):
    o_ref[...] = lax.dot_general(
        x_ref[...], wr_ref[...], _DN_T,
        preferred_element_type=jnp.float32) + b_ref[...]


_tc_root_call = pl.pallas_call(
    _tc_root,
    grid=(N_NODES // _ROW_BLK,),
    in_specs=[
        pl.BlockSpec((_ROW_BLK, D), lambda i: (i, 0)),
        pl.BlockSpec((D, D), lambda i: (0, 0)),
        pl.BlockSpec((1, D), lambda i: (0, 0)),
    ],
    out_specs=pl.BlockSpec((_ROW_BLK, D), lambda i: (i, 0)),
    out_shape=jax.ShapeDtypeStruct((N_NODES, D), jnp.float32),
)


def _tc_fin(p_ref, r_ref, wl_ref, o_ref):
    agg = p_ref[0] + p_ref[1]
    y = lax.dot_general(agg, wl_ref[...], _DN_T,
                        preferred_element_type=jnp.float32)
    o_ref[...] = jnp.tanh(y + r_ref[...])


_tc_fin_call = pl.pallas_call(
    _tc_fin,
    grid=(N_NODES // _ROW_BLK,),
    in_specs=[
        pl.BlockSpec((NC, _ROW_BLK, D), lambda i: (0, i, 0)),
        pl.BlockSpec((_ROW_BLK, D), lambda i: (i, 0)),
        pl.BlockSpec((D, D), lambda i: (0, 0)),
    ],
    out_specs=pl.BlockSpec((_ROW_BLK, D), lambda i: (i, 0)),
    out_shape=jax.ShapeDtypeStruct((N_NODES, D), jnp.float32),
)


def kernel(x, edge_index, W_l, b_l, W_r):
    ei = edge_index.astype(jnp.int32).reshape(-1)
    root = _tc_root_call(x, W_r, b_l.reshape(1, D))
    partials = _sc_aggregate(x, ei)
    return _tc_fin_call(partials, root, W_l)


# speedup vs baseline: 13.6423x; 1.0428x over previous
import functools

import jax
import jax.numpy as jnp
from jax import lax
from jax.experimental import pallas as pl
from jax.experimental.pallas import tpu as pltpu
from jax.experimental.pallas import tpu_sc as plsc

N_NODES = 10000
N_EDGES = 320000
D = 128

NC = 2
NS = 16
NW = NC * NS
EPW = N_EDGES // NW
CHUNK = 80
NCHUNK = EPW // CHUNK
N_PAD = 10112
ROWS_PER_TILE = N_PAD // NS
NBUF = 3

_sc_mesh = plsc.VectorSubcoreMesh(core_axis_name="c", subcore_axis_name="s")


@functools.partial(
    pl.kernel,
    out_type=jax.ShapeDtypeStruct((NC, N_PAD, D), jnp.float32),
    mesh=_sc_mesh,
    scratch_types=[
        pltpu.VMEM((EPW,), jnp.int32),
        pltpu.VMEM((CHUNK,), jnp.int32),
        pltpu.VMEM((CHUNK,), jnp.int32),
        pltpu.VMEM((CHUNK,), jnp.int32),
        pltpu.VMEM((CHUNK, D), jnp.float32),
        pltpu.VMEM((CHUNK, D), jnp.float32),
        pltpu.VMEM((CHUNK, D), jnp.float32),
        pltpu.VMEM_SHARED((N_PAD, D), jnp.float32),
        pltpu.SemaphoreType.DMA,
        pltpu.SemaphoreType.DMA,
        pltpu.SemaphoreType.DMA,
    ],
)
def _sc_aggregate(x_hbm, ei_hbm, out_hbm,
                  dst_v, si0, si1, si2, rb0, rb1, rb2, acc,
                  sem_i, sem_g, sem_s):
    c = lax.axis_index("c")
    s = lax.axis_index("s")
    w = c * NS + s

    sbufs = (si0, si1, si2)
    bufs = (rb0, rb1, rb2)

    def zrow(i, carry):
        for k in range(D // 16):
            rb0[i, pl.ds(k * 16, 16)] = jnp.zeros((16,), jnp.float32)
        return carry

    lax.fori_loop(0, CHUNK, zrow, 0)
    r0 = pl.multiple_of(s * ROWS_PER_TILE, 8)
    for t in range(ROWS_PER_TILE // CHUNK):
        pltpu.sync_copy(rb0, acc.at[pl.ds(r0 + t * CHUNK, CHUNK)])
    rem = ROWS_PER_TILE % CHUNK
    if rem:
        pltpu.sync_copy(
            rb0.at[pl.ds(0, rem)],
            acc.at[pl.ds(r0 + ROWS_PER_TILE - rem, rem)])
    e0 = pl.multiple_of(w * EPW, 8)
    pltpu.sync_copy(ei_hbm.at[pl.ds(N_EDGES + e0, EPW)], dst_v)
    plsc.subcore_barrier()

    def start_idx(j, b):
        pltpu.async_copy(
            ei_hbm.at[pl.ds(e0 + pl.multiple_of(j * CHUNK, 8), CHUNK)],
            sbufs[b], sem_i)

    def wait_idx(b):
        pltpu.make_async_copy(ei_hbm.at[pl.ds(0, CHUNK)], sbufs[b],
                              sem_i).wait()

    def start_gather(j, b):
        del j
        pltpu.async_copy(x_hbm.at[sbufs[b]], bufs[b], sem_g)

    def wait_gather(b):
        pltpu.make_async_copy(x_hbm.at[sbufs[b]], bufs[b], sem_g).wait()

    def start_scatter(j, b):
        pltpu.async_copy(
            bufs[b],
            acc.at[dst_v.at[pl.ds(pl.multiple_of(j * CHUNK, 8), CHUNK)]],
            sem_s, add=True)

    def wait_scatter(b):
        pltpu.make_async_copy(
            bufs[b], acc.at[dst_v.at[pl.ds(0, CHUNK)]], sem_s).wait()

    last = NCHUNK - 1

    def emit_round(j, static):
        for b in range(NBUF):
            cchunk = j + b
            if static and isinstance(cchunk, int) and cchunk > last:
                continue
            wait_gather(b)
            if not static or j + b + NBUF <= last:
                start_idx(j + b + NBUF, b)
            start_scatter(cchunk, b)
        for b in range(NBUF):
            if static and j + b > last:
                continue
            wait_scatter(b)
            if not static or j + b + NBUF <= last:
                wait_idx(b)
                start_gather(j + b + NBUF, b)

    for b in range(NBUF):
        start_idx(b, b)
    for b in range(NBUF):
        wait_idx(b)
        start_gather(b, b)

    def body(i, carry):
        emit_round(3 * i, False)
        return carry

    lax.fori_loop(0, 39, body, 0)
    for j in (117, 120, 123):
        emit_round(j, True)
    plsc.subcore_barrier()

    pltpu.sync_copy(acc.at[pl.ds(r0, ROWS_PER_TILE)],
                    out_hbm.at[c].at[pl.ds(r0, ROWS_PER_TILE)])


_ROW_BLK = 1000
_DN_T = (((1,), (1,)), ((), ()))


def _tc_root(x_ref, wr_ref, b_ref, o_ref):
    o_ref[...] = lax.dot_general(
        x_ref[...], wr_ref[...], _DN_T,
        preferred_element_type=jnp.float32) + b_ref[...]


_tc_root_call = pl.pallas_call(
    _tc_root,
    grid=(N_NODES // _ROW_BLK,),
    in_specs=[
        pl.BlockSpec((_ROW_BLK, D), lambda i: (i, 0)),
        pl.BlockSpec((D, D), lambda i: (0, 0)),
        pl.BlockSpec((1, D), lambda i: (0, 0)),
    ],
    out_specs=pl.BlockSpec((_ROW_BLK, D), lambda i: (i, 0)),
    out_shape=jax.ShapeDtypeStruct((N_NODES, D), jnp.float32),
)


def _tc_fin(p_ref, r_ref, wl_ref, o_ref):
    agg = p_ref[0] + p_ref[1]
    y = lax.dot_general(agg, wl_ref[...], _DN_T,
                        preferred_element_type=jnp.float32)
    o_ref[...] = jnp.tanh(y + r_ref[...])


_tc_fin_call = pl.pallas_call(
    _tc_fin,
    grid=(N_NODES // _ROW_BLK,),
    in_specs=[
        pl.BlockSpec((NC, _ROW_BLK, D), lambda i: (0, i, 0)),
        pl.BlockSpec((_ROW_BLK, D), lambda i: (i, 0)),
        pl.BlockSpec((D, D), lambda i: (0, 0)),
    ],
    out_specs=pl.BlockSpec((_ROW_BLK, D), lambda i: (i, 0)),
    out_shape=jax.ShapeDtypeStruct((N_NODES, D), jnp.float32),
)


def kernel(x, edge_index, W_l, b_l, W_r):
    ei = edge_index.astype(jnp.int32).reshape(-1)
    root = _tc_root_call(x, W_r, b_l.reshape(1, D))
    partials = _sc_aggregate(x, ei)
    return _tc_fin_call(partials, root, W_l)


# speedup vs baseline: 14.0867x; 1.0326x over previous
import functools

import jax
import jax.numpy as jnp
from jax import lax
from jax.experimental import pallas as pl
from jax.experimental.pallas import tpu as pltpu
from jax.experimental.pallas import tpu_sc as plsc

N_NODES = 10000
N_EDGES = 320000
D = 128

NC = 2
NS = 16
NW = NC * NS
EPW = N_EDGES // NW
CHUNK = 128
NCHUNK = EPW // CHUNK
TAIL = EPW - NCHUNK * CHUNK
N_PAD = 10112
ROWS_PER_TILE = N_PAD // NS
NBUF = 3

_sc_mesh = plsc.VectorSubcoreMesh(core_axis_name="c", subcore_axis_name="s")


@functools.partial(
    pl.kernel,
    out_type=jax.ShapeDtypeStruct((NC, N_PAD, D), jnp.float32),
    mesh=_sc_mesh,
    scratch_types=[
        pltpu.VMEM((CHUNK,), jnp.int32),
        pltpu.VMEM((CHUNK,), jnp.int32),
        pltpu.VMEM((CHUNK,), jnp.int32),
        pltpu.VMEM((CHUNK,), jnp.int32),
        pltpu.VMEM((CHUNK,), jnp.int32),
        pltpu.VMEM((CHUNK,), jnp.int32),
        pltpu.VMEM((CHUNK, D), jnp.float32),
        pltpu.VMEM((CHUNK, D), jnp.float32),
        pltpu.VMEM((CHUNK, D), jnp.float32),
        pltpu.VMEM_SHARED((N_PAD, D), jnp.float32),
        pltpu.SemaphoreType.DMA,
        pltpu.SemaphoreType.DMA,
        pltpu.SemaphoreType.DMA,
    ],
)
def _sc_aggregate(x_hbm, ei_hbm, out_hbm,
                  si0, si1, si2, di0, di1, di2, rb0, rb1, rb2, acc,
                  sem_i, sem_g, sem_s):
    c = lax.axis_index("c")
    s = lax.axis_index("s")
    w = c * NS + s

    sbufs = (si0, si1, si2)
    dbufs = (di0, di1, di2)
    bufs = (rb0, rb1, rb2)

    def zrow(i, carry):
        for k in range(D // 16):
            rb0[i, pl.ds(k * 16, 16)] = jnp.zeros((16,), jnp.float32)
        return carry

    lax.fori_loop(0, CHUNK, zrow, 0)
    r0 = pl.multiple_of(s * ROWS_PER_TILE, 8)
    for t in range(ROWS_PER_TILE // CHUNK):
        pltpu.sync_copy(rb0, acc.at[pl.ds(r0 + t * CHUNK, CHUNK)])
    rem = ROWS_PER_TILE % CHUNK
    if rem:
        pltpu.sync_copy(
            rb0.at[pl.ds(0, rem)],
            acc.at[pl.ds(r0 + ROWS_PER_TILE - rem, rem)])
    e0 = pl.multiple_of(w * EPW, 8)
    plsc.subcore_barrier()

    def start_idx(j, b):
        off = pl.multiple_of(j * CHUNK, 8)
        pltpu.async_copy(ei_hbm.at[pl.ds(e0 + off, CHUNK)], sbufs[b], sem_i)
        pltpu.async_copy(ei_hbm.at[pl.ds(N_EDGES + e0 + off, CHUNK)],
                         dbufs[b], sem_i)

    def wait_idx(b):
        pltpu.make_async_copy(ei_hbm.at[pl.ds(0, CHUNK)], sbufs[b],
                              sem_i).wait()
        pltpu.make_async_copy(ei_hbm.at[pl.ds(0, CHUNK)], dbufs[b],
                              sem_i).wait()

    def start_gather(j, b):
        del j
        pltpu.async_copy(x_hbm.at[sbufs[b]], bufs[b], sem_g)

    def wait_gather(b):
        pltpu.make_async_copy(x_hbm.at[sbufs[b]], bufs[b], sem_g).wait()

    def start_scatter(j, b):
        del j
        pltpu.async_copy(bufs[b], acc.at[dbufs[b]], sem_s, add=True)

    def wait_scatter(b):
        pltpu.make_async_copy(bufs[b], acc.at[dbufs[b]], sem_s).wait()

    last = NCHUNK - 1

    def emit_round(j, static):
        for b in range(NBUF):
            cchunk = j + b
            if static and isinstance(cchunk, int) and cchunk > last:
                continue
            wait_gather(b)
            if not static or j + b + NBUF <= last:
                start_idx(j + b + NBUF, b)
            start_scatter(cchunk, b)
        for b in range(NBUF):
            if static and j + b > last:
                continue
            wait_scatter(b)
            if not static or j + b + NBUF <= last:
                wait_idx(b)
                start_gather(j + b + NBUF, b)

    for b in range(NBUF):
        start_idx(b, b)
    for b in range(NBUF):
        wait_idx(b)
        start_gather(b, b)

    def body(i, carry):
        emit_round(3 * i, False)
        return carry

    lax.fori_loop(0, 25, body, 0)
    emit_round(75, True)
    toff = pl.multiple_of(NCHUNK * CHUNK, 8)
    pltpu.async_copy(ei_hbm.at[pl.ds(e0 + toff, TAIL)],
                     si0.at[pl.ds(0, TAIL)], sem_i)
    pltpu.async_copy(ei_hbm.at[pl.ds(N_EDGES + e0 + toff, TAIL)],
                     di0.at[pl.ds(0, TAIL)], sem_i)
    pltpu.make_async_copy(ei_hbm.at[pl.ds(0, TAIL)],
                          si0.at[pl.ds(0, TAIL)], sem_i).wait()
    pltpu.make_async_copy(ei_hbm.at[pl.ds(0, TAIL)],
                          di0.at[pl.ds(0, TAIL)], sem_i).wait()
    pltpu.async_copy(x_hbm.at[si0.at[pl.ds(0, TAIL)]],
                     rb0.at[pl.ds(0, TAIL)], sem_g)
    pltpu.make_async_copy(x_hbm.at[si0.at[pl.ds(0, TAIL)]],
                          rb0.at[pl.ds(0, TAIL)], sem_g).wait()
    pltpu.sync_copy(rb0.at[pl.ds(0, TAIL)],
                    acc.at[di0.at[pl.ds(0, TAIL)]], add=True)
    plsc.subcore_barrier()

    pltpu.sync_copy(acc.at[pl.ds(r0, ROWS_PER_TILE)],
                    out_hbm.at[c].at[pl.ds(r0, ROWS_PER_TILE)])


_ROW_BLK = 1000
_DN_T = (((1,), (1,)), ((), ()))


def _tc_root(x_ref, wr_ref, b_ref, o_ref):
    o_ref[...] = lax.dot_general(
        x_ref[...], wr_ref[...], _DN_T,
        preferred_element_type=jnp.float32) + b_ref[...]


_tc_root_call = pl.pallas_call(
    _tc_root,
    grid=(N_NODES // _ROW_BLK,),
    in_specs=[
        pl.BlockSpec((_ROW_BLK, D), lambda i: (i, 0)),
        pl.BlockSpec((D, D), lambda i: (0, 0)),
        pl.BlockSpec((1, D), lambda i: (0, 0)),
    ],
    out_specs=pl.BlockSpec((_ROW_BLK, D), lambda i: (i, 0)),
    out_shape=jax.ShapeDtypeStruct((N_NODES, D), jnp.float32),
)


def _tc_fin(p_ref, r_ref, wl_ref, o_ref):
    agg = p_ref[0] + p_ref[1]
    y = lax.dot_general(agg, wl_ref[...], _DN_T,
                        preferred_element_type=jnp.float32)
    o_ref[...] = jnp.tanh(y + r_ref[...])


_tc_fin_call = pl.pallas_call(
    _tc_fin,
    grid=(N_NODES // _ROW_BLK,),
    in_specs=[
        pl.BlockSpec((NC, _ROW_BLK, D), lambda i: (0, i, 0)),
        pl.BlockSpec((_ROW_BLK, D), lambda i: (i, 0)),
        pl.BlockSpec((D, D), lambda i: (0, 0)),
    ],
    out_specs=pl.BlockSpec((_ROW_BLK, D), lambda i: (i, 0)),
    out_shape=jax.ShapeDtypeStruct((N_NODES, D), jnp.float32),
)


def kernel(x, edge_index, W_l, b_l, W_r):
    ei = edge_index.astype(jnp.int32).reshape(-1)
    root = _tc_root_call(x, W_r, b_l.reshape(1, D))
    partials = _sc_aggregate(x, ei)
    return _tc_fin_call(partials, root, W_l)
